# Initial kernel scaffold; baseline (speedup 1.0000x reference)
import jax
import jax.numpy as jnp
from jax.experimental import pallas as pl


def kernel(feature_vectors, W1, b1, W2, b2, Wc, bc, Wr, br):
    raise NotImplementedError("write your pallas kernel here")


# speedup vs baseline: 1.0161x; 1.0161x over previous
import jax
import jax.numpy as jnp
from jax.experimental import pallas as pl
from jax.experimental.pallas import tpu as pltpu

_N = 5000
_D = 12544
_H = 1024
_BM = 1000
_BK = 1792
_NK = _D // _BK


def _mlp_body(feat_ref, w1_ref, b1_ref, w2_ref, b2_ref, wh_ref, bh_ref,
              out_ref, acc_ref):
    k = pl.program_id(1)

    @pl.when(k == 0)
    def _init():
        acc_ref[...] = jnp.zeros_like(acc_ref)

    acc_ref[...] += jnp.dot(feat_ref[...], w1_ref[...],
                            preferred_element_type=jnp.float32)

    @pl.when(k == _NK - 1)
    def _final():
        h1 = jnp.maximum(acc_ref[...] + b1_ref[...], 0.0)
        h2 = jnp.maximum(
            jnp.dot(h1, w2_ref[...], preferred_element_type=jnp.float32)
            + b2_ref[...], 0.0)
        out_ref[...] = (
            jnp.dot(h2, wh_ref[...], preferred_element_type=jnp.float32)
            + bh_ref[...])


def kernel(feature_vectors, W1, b1, W2, b2, Wc, bc, Wr, br):
    Wh = jnp.concatenate([Wc, Wr], axis=1)
    bh = jnp.concatenate([bc, br])[None, :]
    out = pl.pallas_call(
        _mlp_body,
        grid=(_N // _BM, _NK),
        in_specs=[
            pl.BlockSpec((_BM, _BK), lambda m, k: (m, k)),
            pl.BlockSpec((_BK, _H), lambda m, k: (k, 0)),
            pl.BlockSpec((1, _H), lambda m, k: (0, 0)),
            pl.BlockSpec((_H, _H), lambda m, k: (0, 0)),
            pl.BlockSpec((1, _H), lambda m, k: (0, 0)),
            pl.BlockSpec((_H, 16), lambda m, k: (0, 0)),
            pl.BlockSpec((1, 16), lambda m, k: (0, 0)),
        ],
        out_specs=pl.BlockSpec((_BM, 16), lambda m, k: (m, 0)),
        out_shape=jax.ShapeDtypeStruct((_N, 16), jnp.float32),
        scratch_shapes=[pltpu.VMEM((_BM, _H), jnp.float32)],
        compiler_params=pltpu.CompilerParams(
            dimension_semantics=("parallel", "arbitrary"),
        ),
    )(feature_vectors, W1, b1[None, :], W2, b2[None, :], Wh, bh)
    return out[:, :4], out[:, 4:]
